# Initial kernel scaffold; baseline (speedup 1.0000x reference)
import jax
import jax.numpy as jnp
from jax.experimental import pallas as pl


def kernel(x0, x1, x2, x3, x4, x5, reg_w0, reg_w1, reg_w2, reg_w3, reg_w4, reg_w5, reg_b0, reg_b1, reg_b2, reg_b3, reg_b4, reg_b5, cls_w0, cls_w1, cls_w2, cls_w3, cls_w4, cls_w5, cls_b0, cls_b1, cls_b2, cls_b3, cls_b4, cls_b5):
    raise NotImplementedError("write your pallas kernel here")


# speedup vs baseline: 1.2736x; 1.2736x over previous
import functools

import jax
import jax.numpy as jnp
from jax.experimental import pallas as pl
from jax.experimental.pallas import tpu as pltpu

_IN_CHANNELS = [512, 1024, 512, 256, 256, 256]
_NUM_ANCHORS = [4, 6, 6, 6, 4, 4]
_NUM_CLASSES = 91
_FEAT_HW = [64, 32, 16, 8, 4, 2]


def _head_kernel(x_ref, w_ref, b_ref, cls_ref, reg_ref, *, H, W, A):
    HW = H * W
    P = W + 1
    ncls = _NUM_CLASSES * A
    total = None
    for dx in (-1, 0, 1):
        acc = None
        for dy in (-1, 0, 1):
            t = (dy + 1) * 3 + (dx + 1)
            xs = x_ref[0, pl.ds(P + dy * W + dx, HW), :]
            m = jnp.dot(xs, w_ref[t], preferred_element_type=jnp.float32)
            acc = m if acc is None else acc + m
        if dx != 0:
            col = jax.lax.broadcasted_iota(jnp.int32, (HW, 1), 0) % W
            bad = col == (0 if dx == -1 else W - 1)
            acc = jnp.where(bad, 0.0, acc)
        total = acc if total is None else total + acc
    total = total + b_ref[...]
    cls_ref[0] = total[:, :ncls]
    reg_ref[0] = total[:, ncls:]


def _level_call(xp, w, b, H, W, A):
    N, L, C = xp.shape
    HW = H * W
    Cout = (_NUM_CLASSES + 4) * A
    kern = functools.partial(_head_kernel, H=H, W=W, A=A)
    return pl.pallas_call(
        kern,
        grid=(N,),
        in_specs=[
            pl.BlockSpec((1, L, C), lambda n: (n, 0, 0)),
            pl.BlockSpec((9, C, Cout), lambda n: (0, 0, 0)),
            pl.BlockSpec((1, Cout), lambda n: (0, 0)),
        ],
        out_specs=[
            pl.BlockSpec((1, HW, _NUM_CLASSES * A), lambda n: (n, 0, 0)),
            pl.BlockSpec((1, HW, 4 * A), lambda n: (n, 0, 0)),
        ],
        out_shape=[
            jax.ShapeDtypeStruct((N, HW, _NUM_CLASSES * A), jnp.float32),
            jax.ShapeDtypeStruct((N, HW, 4 * A), jnp.float32),
        ],
        compiler_params=pltpu.CompilerParams(
            dimension_semantics=("arbitrary",),
        ),
    )(xp, w, b)


def kernel(x0, x1, x2, x3, x4, x5, reg_w0, reg_w1, reg_w2, reg_w3, reg_w4, reg_w5, reg_b0, reg_b1, reg_b2, reg_b3, reg_b4, reg_b5, cls_w0, cls_w1, cls_w2, cls_w3, cls_w4, cls_w5, cls_b0, cls_b1, cls_b2, cls_b3, cls_b4, cls_b5):
    xs = [x0, x1, x2, x3, x4, x5]
    reg_ws = [reg_w0, reg_w1, reg_w2, reg_w3, reg_w4, reg_w5]
    reg_bs = [reg_b0, reg_b1, reg_b2, reg_b3, reg_b4, reg_b5]
    cls_ws = [cls_w0, cls_w1, cls_w2, cls_w3, cls_w4, cls_w5]
    cls_bs = [cls_b0, cls_b1, cls_b2, cls_b3, cls_b4, cls_b5]

    cls_parts, reg_parts = [], []
    for i in range(6):
        C = _IN_CHANNELS[i]
        A = _NUM_ANCHORS[i]
        H = W = _FEAT_HW[i]
        HW = H * W
        Cout = (_NUM_CLASSES + 4) * A
        N = xs[i].shape[0]

        xt = jnp.transpose(xs[i].reshape(N, C, HW), (0, 2, 1))
        xt = xt.astype(jnp.bfloat16)
        xp = jnp.pad(xt, ((0, 0), (W + 1, W + 1), (0, 0)))

        wc = jnp.concatenate([cls_ws[i], reg_ws[i]], axis=0)
        wc = jnp.transpose(wc, (2, 3, 1, 0)).reshape(9, C, Cout)
        wc = wc.astype(jnp.bfloat16)
        bc = jnp.concatenate([cls_bs[i], reg_bs[i]])[None, :]

        cls_i, reg_i = _level_call(xp, wc, bc, H, W, A)
        cls_parts.append(cls_i.reshape(N, HW * A, _NUM_CLASSES))
        reg_parts.append(reg_i.reshape(N, HW * A, 4))

    bbox_regression = jnp.concatenate(reg_parts, axis=1)
    cls_logits = jnp.concatenate(cls_parts, axis=1)
    return (bbox_regression, cls_logits)
